# Initial kernel scaffold; baseline (speedup 1.0000x reference)
import jax
import jax.numpy as jnp
from jax.experimental import pallas as pl


def kernel(tokens, positions, x):
    raise NotImplementedError("write your pallas kernel here")


# speedup vs baseline: 1.4977x; 1.4977x over previous
import functools

import jax
import jax.numpy as jnp
from jax import lax
from jax.experimental import pallas as pl
from jax.experimental.pallas import tpu as pltpu
from jax.experimental.pallas import tpu_sc as plsc

LANES = 16
CHUNK = 100


@functools.lru_cache(maxsize=None)
def _build(num_rows, seq_len, dim, num_chunks):
  info = plsc.get_sparse_core_info()
  nc, ns = info.num_cores, info.num_subcores
  nw = nc * ns
  cpw = num_chunks // nw

  mesh = plsc.VectorSubcoreMesh(core_axis_name="c", subcore_axis_name="s")

  @functools.partial(
      pl.kernel,
      mesh=mesh,
      out_type=jax.ShapeDtypeStruct((num_chunks, CHUNK, dim), jnp.float32),
      scratch_types=[
          pltpu.VMEM((cpw, CHUNK), jnp.int32),
          pltpu.VMEM((seq_len, dim), jnp.float32),
          pltpu.VMEM((CHUNK, dim), jnp.float32),
          pltpu.SemaphoreType.DMA,
      ],
  )
  def emb(tokens_hbm, pos_hbm, x_hbm, out_hbm, idx_v, pos_v, rows_v, sem):
    wid = lax.axis_index("s") * nc + lax.axis_index("c")
    base = wid * cpw
    pltpu.sync_copy(x_hbm.at[pl.ds(base, cpw)], idx_v)
    pltpu.sync_copy(pos_hbm, pos_v)

    def chunk_body(c, carry):
      pltpu.async_copy(tokens_hbm.at[idx_v.at[c]], rows_v, sem).wait()
      off = (c % 2) * CHUNK

      def row_body(i, rcarry):
        for j in range(dim // LANES):
          sl = pl.ds(j * LANES, LANES)
          rows_v[i, sl] = rows_v[i, sl] + pos_v[off + i, sl]
        return rcarry

      lax.fori_loop(0, CHUNK, row_body, 0)
      pltpu.sync_copy(rows_v, out_hbm.at[base + c])
      return carry

    lax.fori_loop(0, cpw, chunk_body, 0)

  return emb


def kernel(tokens, positions, x):
  b, s = x.shape
  _, dim = tokens.shape
  num_rows = b * s
  num_chunks = num_rows // CHUNK
  x2 = x.reshape(num_chunks, CHUNK)
  pos = positions[:s]
  out = _build(num_rows, s, dim, num_chunks)(tokens, pos, x2)
  return out.reshape(b, s, dim)


# speedup vs baseline: 3.3522x; 2.2382x over previous
import functools

import jax
import jax.numpy as jnp
from jax import lax
from jax.experimental import pallas as pl
from jax.experimental.pallas import tpu as pltpu
from jax.experimental.pallas import tpu_sc as plsc

LANES = 16
CHUNK = 100
NBUF = 4


@functools.lru_cache(maxsize=None)
def _build(seq_len, dim, num_chunks):
  info = plsc.get_sparse_core_info()
  nc, ns = info.num_cores, info.num_subcores
  nw = nc * ns
  cpw = num_chunks // nw
  nt = cpw // NBUF

  mesh = plsc.VectorSubcoreMesh(core_axis_name="c", subcore_axis_name="s")

  @functools.partial(
      pl.kernel,
      mesh=mesh,
      out_type=jax.ShapeDtypeStruct((num_chunks, CHUNK, dim), jnp.float32),
      scratch_types=[
          pltpu.VMEM((cpw, CHUNK), jnp.int32),
          pltpu.VMEM((seq_len, dim), jnp.float32),
          pltpu.VMEM((NBUF, CHUNK, dim), jnp.float32),
          pltpu.SemaphoreType.DMA((NBUF,)),
          pltpu.SemaphoreType.DMA((NBUF,)),
      ],
  )
  def emb(tokens_hbm, pos_hbm, x_hbm, out_hbm, idx_v, pos_v, rows_v,
          gsem, wsem):
    wid = lax.axis_index("s") * nc + lax.axis_index("c")
    base = wid * cpw
    pltpu.sync_copy(x_hbm.at[pl.ds(base, cpw)], idx_v)
    pltpu.sync_copy(pos_hbm, pos_v)

    def start_gather(c, b):
      pltpu.async_copy(tokens_hbm.at[idx_v.at[c]], rows_v.at[b], gsem.at[b])

    def wait_gather(b):
      pltpu.make_async_copy(
          out_hbm.at[0], rows_v.at[b], gsem.at[b]).wait()

    def start_wb(c, b):
      pltpu.async_copy(rows_v.at[b], out_hbm.at[base + c], wsem.at[b])

    def wait_wb(b):
      pltpu.make_async_copy(
          rows_v.at[b], out_hbm.at[0], wsem.at[b]).wait()

    def add_pos(b):
      off = (b % 2) * CHUNK

      def row_body(i, rcarry):
        for j in range(dim // LANES):
          sl = pl.ds(j * LANES, LANES)
          rows_v[b, i, sl] = rows_v[b, i, sl] + pos_v[off + i, sl]
        return rcarry

      lax.fori_loop(0, CHUNK, row_body, 0)

    for b in range(NBUF):
      start_gather(b, b)

    def outer(t, carry):
      c0 = t * NBUF
      for b in range(NBUF):
        wait_gather(b)
        add_pos(b)
        start_wb(c0 + b, b)
      for b in range(NBUF):
        wait_wb(b)
        start_gather(c0 + NBUF + b, b)
      return carry

    lax.fori_loop(0, nt - 1, outer, 0)

    c0 = (nt - 1) * NBUF
    for b in range(NBUF):
      wait_gather(b)
      add_pos(b)
      start_wb(c0 + b, b)
    for b in range(NBUF):
      wait_wb(b)

  return emb


def kernel(tokens, positions, x):
  b, s = x.shape
  _, dim = tokens.shape
  num_chunks = b * s // CHUNK
  x2 = x.reshape(num_chunks, CHUNK)
  pos = positions[:s]
  out = _build(s, dim, num_chunks)(tokens, pos, x2)
  return out.reshape(b, s, dim)


# speedup vs baseline: 6.1483x; 1.8341x over previous
import functools

import jax
import jax.numpy as jnp
from jax import lax
from jax.experimental import pallas as pl
from jax.experimental.pallas import tpu as pltpu
from jax.experimental.pallas import tpu_sc as plsc

LANES = 16
CHUNK = 100
NBUF = 2


@functools.lru_cache(maxsize=None)
def _build(batch, seq_len, dim):
  info = plsc.get_sparse_core_info()
  nc, ns = info.num_cores, info.num_subcores
  nw = nc * ns
  spw = batch // nw
  cps = seq_len // CHUNK
  nt = spw // NBUF

  mesh = plsc.VectorSubcoreMesh(core_axis_name="c", subcore_axis_name="s")

  @functools.partial(
      pl.kernel,
      mesh=mesh,
      out_type=jax.ShapeDtypeStruct((batch, seq_len, dim), jnp.float32),
      scratch_types=[
          pltpu.VMEM((spw * cps, CHUNK), jnp.int32),
          pltpu.VMEM((seq_len, dim), jnp.float32),
          pltpu.VMEM((NBUF, seq_len, dim), jnp.float32),
          pltpu.SemaphoreType.DMA((NBUF,)),
          pltpu.SemaphoreType.DMA((NBUF,)),
      ],
  )
  def emb(tokens_hbm, pos_hbm, x_hbm, out_hbm, idx_v, pos_v, rows_v,
          gsem, wsem):
    wid = lax.axis_index("s") * nc + lax.axis_index("c")
    base = wid * spw
    pltpu.sync_copy(x_hbm.at[pl.ds(base * cps, spw * cps)], idx_v)
    pltpu.sync_copy(pos_hbm, pos_v)

    def start_gather(q, b):
      for h in range(cps):
        pltpu.async_copy(
            tokens_hbm.at[idx_v.at[q * cps + h]],
            rows_v.at[b, pl.ds(h * CHUNK, CHUNK)],
            gsem.at[b])

    def wait_gather(b):
      pltpu.make_async_copy(out_hbm.at[0], rows_v.at[b], gsem.at[b]).wait()

    def start_wb(q, b):
      pltpu.async_copy(rows_v.at[b], out_hbm.at[base + q], wsem.at[b])

    def wait_wb(b):
      pltpu.make_async_copy(rows_v.at[b], out_hbm.at[0], wsem.at[b]).wait()

    def add_pos(b):
      def row_body(i, rcarry):
        for u in range(2):
          for j in range(dim // LANES):
            sl = pl.ds(j * LANES, LANES)
            rows_v[b, 2 * i + u, sl] = (
                rows_v[b, 2 * i + u, sl] + pos_v[2 * i + u, sl])
        return rcarry

      lax.fori_loop(0, seq_len // 2, row_body, 0)

    for b in range(NBUF):
      start_gather(b, b)

    def outer(t, carry):
      q0 = t * NBUF
      for b in range(NBUF):
        wait_gather(b)
        add_pos(b)
        start_wb(q0 + b, b)
      for b in range(NBUF):
        wait_wb(b)
        start_gather(q0 + NBUF + b, b)
      return carry

    lax.fori_loop(0, nt - 1, outer, 0)

    q0 = (nt - 1) * NBUF
    for b in range(NBUF):
      wait_gather(b)
      add_pos(b)
      start_wb(q0 + b, b)
    for b in range(NBUF):
      wait_wb(b)

  return emb


def kernel(tokens, positions, x):
  b, s = x.shape
  _, dim = tokens.shape
  x2 = x.reshape(b * s // CHUNK, CHUNK)
  pos = positions[:s]
  return _build(b, s, dim)(tokens, pos, x2)


# speedup vs baseline: 6.2231x; 1.0122x over previous
import functools

import jax
import jax.numpy as jnp
from jax import lax
from jax.experimental import pallas as pl
from jax.experimental.pallas import tpu as pltpu
from jax.experimental.pallas import tpu_sc as plsc

LANES = 16
CHUNK = 100
NBUF = 2


@functools.lru_cache(maxsize=None)
def _build(batch, seq_len, dim):
  info = plsc.get_sparse_core_info()
  nc, ns = info.num_cores, info.num_subcores
  nw = nc * ns
  spw = batch // nw
  cps = seq_len // CHUNK
  nt = spw // NBUF

  mesh = plsc.VectorSubcoreMesh(core_axis_name="c", subcore_axis_name="s")

  @functools.partial(
      pl.kernel,
      mesh=mesh,
      out_type=jax.ShapeDtypeStruct((batch, seq_len, dim), jnp.float32),
      scratch_types=[
          pltpu.VMEM((spw * cps, CHUNK), jnp.int32),
          pltpu.VMEM((seq_len, dim), jnp.float32),
          pltpu.VMEM((NBUF, seq_len, dim), jnp.float32),
          pltpu.SemaphoreType.DMA((NBUF,)),
          pltpu.SemaphoreType.DMA((NBUF,)),
      ],
  )
  def emb(tokens_hbm, pos_hbm, x_hbm, out_hbm, idx_v, pos_v, rows_v,
          gsem, wsem):
    wid = lax.axis_index("s") * nc + lax.axis_index("c")
    base = wid * spw
    pltpu.sync_copy(x_hbm.at[pl.ds(base * cps, spw * cps)], idx_v)
    pltpu.sync_copy(pos_hbm.at[pl.ds(0, seq_len)], pos_v)

    def start_gather(q, b):
      for h in range(cps):
        pltpu.async_copy(
            tokens_hbm.at[idx_v.at[q * cps + h]],
            rows_v.at[b, pl.ds(h * CHUNK, CHUNK)],
            gsem.at[b])

    def wait_gather(b):
      pltpu.make_async_copy(out_hbm.at[0], rows_v.at[b], gsem.at[b]).wait()

    def start_wb(q, b):
      pltpu.async_copy(rows_v.at[b], out_hbm.at[base + q], wsem.at[b])

    def wait_wb(b):
      pltpu.make_async_copy(rows_v.at[b], out_hbm.at[0], wsem.at[b]).wait()

    def add_pos(b):
      def row_body(i, rcarry):
        for u in range(2):
          for j in range(dim // LANES):
            sl = pl.ds(j * LANES, LANES)
            rows_v[b, 2 * i + u, sl] = (
                rows_v[b, 2 * i + u, sl] + pos_v[2 * i + u, sl])
        return rcarry

      lax.fori_loop(0, seq_len // 2, row_body, 0)

    for b in range(NBUF):
      start_gather(b, b)

    def outer(t, carry):
      q0 = t * NBUF
      for b in range(NBUF):
        wait_gather(b)
        add_pos(b)
        start_wb(q0 + b, b)
      for b in range(NBUF):
        wait_wb(b)
        start_gather(q0 + NBUF + b, b)
      return carry

    lax.fori_loop(0, nt - 1, outer, 0)

    q0 = (nt - 1) * NBUF
    for b in range(NBUF):
      wait_gather(b)
      add_pos(b)
      start_wb(q0 + b, b)
    for b in range(NBUF):
      wait_wb(b)

  return emb


def kernel(tokens, positions, x):
  b, s = x.shape
  _, dim = tokens.shape
  x2 = x.reshape(b * s // CHUNK, CHUNK)
  return _build(b, s, dim)(tokens, positions, x2)
